# baseline (speedup 1.0000x reference)
import functools

import jax
import jax.numpy as jnp
from jax import lax
from jax.experimental import pallas as pl
from jax.experimental.pallas import tpu as pltpu
from jax.experimental.pallas import tpu_sc as plsc

N = 10000
H = 128
E = 320000
NC = 2
NS = 16
NW = NC * NS
EPW = E // NW
CHUNK = 128
NCHUNK = 80
EPP = NCHUNK * CHUNK
PAD_ROW = N
NACC = N + 16
RBUF = 2
SBUF = 4
NOUTER = NCHUNK // SBUF

RPT = (N // NS) // 8 * 8
RTAIL = N - RPT * NS

assert EPW * NW == E and EPP >= EPW
assert NOUTER * SBUF == NCHUNK and RTAIL % 8 == 0


def _make_sc_agg():
    mesh = plsc.VectorSubcoreMesh(core_axis_name="c", subcore_axis_name="s",
                                  num_cores=NC, num_subcores=NS)

    @functools.partial(
        pl.kernel,
        out_type=jax.ShapeDtypeStruct((NC, N, H), jnp.float32),
        mesh=mesh,
        scratch_types=(
            [
                pltpu.VMEM_SHARED((NACC, H), jnp.float32),
                pltpu.VMEM((SBUF, CHUNK), jnp.int32),
                pltpu.VMEM((NCHUNK, CHUNK), jnp.int32),
                pltpu.VMEM((RBUF, CHUNK, H), jnp.float32),
            ]
            + [pltpu.SemaphoreType.DMA for _ in range(SBUF + 2 * RBUF)]
        ),
    )
    def sc_agg(x_hbm, src_hbm, dst_hbm, zeros_hbm, out_hbm,
               acc_sh, sidx_v, dst_v, rows_v, *sems):
        isem = sems[:SBUF]
        gsem = sems[SBUF:SBUF + RBUF]
        ssem = sems[SBUF + RBUF:]
        c = lax.axis_index("c")
        s = lax.axis_index("s")
        wid = s * NC + c

        pltpu.sync_copy(zeros_hbm.at[pl.ds(0, RPT)],
                        acc_sh.at[pl.ds(s * RPT, RPT)])

        @pl.when(s == NS - 1)
        def _():
            pltpu.sync_copy(zeros_hbm.at[pl.ds(0, RTAIL)],
                            acc_sh.at[pl.ds(NS * RPT, RTAIL)])

        pltpu.sync_copy(dst_hbm.at[wid], dst_v)
        plsc.subcore_barrier()

        def start_idx(j, sb):
            pltpu.async_copy(src_hbm.at[wid, j], sidx_v.at[sb], isem[sb])

        def wait_idx(sb):
            pltpu.make_async_copy(src_hbm.at[wid, 0], sidx_v.at[sb],
                                  isem[sb]).wait()

        def start_gather(sb, rb):
            pltpu.async_copy(x_hbm.at[sidx_v.at[sb]], rows_v.at[rb], gsem[rb])

        def wait_gather(rb):
            pltpu.make_async_copy(x_hbm.at[sidx_v.at[0]], rows_v.at[rb],
                                  gsem[rb]).wait()

        def start_scatter(j, rb):
            pltpu.async_copy(rows_v.at[rb], acc_sh.at[dst_v.at[j]], ssem[rb],
                             add=True)

        def wait_scatter(rb):
            pltpu.make_async_copy(rows_v.at[rb], acc_sh.at[dst_v.at[0]],
                                  ssem[rb]).wait()

        for j in range(SBUF - 1):
            start_idx(j, j)
        wait_idx(0)
        start_gather(0, 0)

        def outer(o, carry):
            for b in range(SBUF):
                j = o * SBUF + b

                @pl.when(j + SBUF - 1 < NCHUNK)
                def _():
                    start_idx(j + SBUF - 1, (b + SBUF - 1) % SBUF)

                @pl.when(j + 1 < NCHUNK)
                def _():
                    wait_idx((b + 1) % SBUF)

                    @pl.when(j - 1 >= 0)
                    def _():
                        wait_scatter((b + 1) % RBUF)

                    start_gather((b + 1) % SBUF, (b + 1) % RBUF)

                wait_gather(b % RBUF)
                start_scatter(j, b % RBUF)
            return carry

        lax.fori_loop(0, NOUTER, outer, 0, unroll=False)

        for rb in range(RBUF):
            wait_scatter(rb)

        plsc.subcore_barrier()
        pltpu.sync_copy(acc_sh.at[pl.ds(s * RPT, RPT)],
                        out_hbm.at[c, pl.ds(s * RPT, RPT)])

        @pl.when(s == NS - 1)
        def _():
            pltpu.sync_copy(acc_sh.at[pl.ds(NS * RPT, RTAIL)],
                            out_hbm.at[c, pl.ds(NS * RPT, RTAIL)])

    return sc_agg


_SC_AGG_CACHE = []


def _sc_agg(*args):
    if not _SC_AGG_CACHE:
        _SC_AGG_CACHE.append(_make_sc_agg())
    return _SC_AGG_CACHE[0](*args)


def _tc_head_body(eps_ref, x_ref, agg_ref, w1_ref, b1_ref, g1_ref, be1_ref,
                  w2_ref, b2_ref, g4_ref, be4_ref, wl1_ref, bl1_ref,
                  wl3_ref, bl3_ref, out_ref):
    eps = eps_ref[0, 0]
    x = x_ref[...]
    agg = agg_ref[0] + agg_ref[1]

    h = (1.0 + eps) * x + agg
    h = jnp.dot(h, w1_ref[...], preferred_element_type=jnp.float32) + b1_ref[...]
    m = jnp.mean(h, axis=0, keepdims=True)
    v = jnp.mean((h - m) * (h - m), axis=0, keepdims=True)
    h = g1_ref[...] * (h - m) * lax.rsqrt(v + 1e-5) + be1_ref[...]
    h = jnp.maximum(h, 0.0)
    h = jnp.dot(h, w2_ref[...], preferred_element_type=jnp.float32) + b2_ref[...]
    h = jnp.where(h > 0, h, 0.01 * h)
    m4 = jnp.mean(h, axis=0, keepdims=True)
    v4 = jnp.mean((h - m4) * (h - m4), axis=0, keepdims=True)
    h = g4_ref[...] * (h - m4) * lax.rsqrt(v4 + 1e-5) + be4_ref[...]
    h = jnp.where(h > 0, h, 0.1 * h)
    h = x + 0.01 * h
    h = jnp.dot(h, wl1_ref[...], preferred_element_type=jnp.float32) + bl1_ref[...]
    h = jnp.where(h > 0, h, 0.1 * h)
    out_ref[...] = (jnp.dot(h, wl3_ref[...], preferred_element_type=jnp.float32)
                    + bl3_ref[...])


def _tc_head(gin_eps, x, agg2, W1, b1, gamma1, beta1, W2, b2, gamma4, beta4,
             Wl1, bl1, Wl3, bl3):
    C = Wl3.shape[1]
    eps_arr = jnp.reshape(gin_eps, (1, 1))
    smem_spec = pl.BlockSpec(memory_space=pltpu.SMEM)
    return pl.pallas_call(
        _tc_head_body,
        out_shape=jax.ShapeDtypeStruct((N, C), jnp.float32),
        in_specs=[smem_spec] + [pl.BlockSpec(memory_space=pltpu.VMEM)] * 14,
        out_specs=pl.BlockSpec(memory_space=pltpu.VMEM),
    )(eps_arr, x, agg2,
      W1, jnp.reshape(b1, (1, H)), jnp.reshape(gamma1, (1, H)),
      jnp.reshape(beta1, (1, H)),
      W2, jnp.reshape(b2, (1, H)), jnp.reshape(gamma4, (1, H)),
      jnp.reshape(beta4, (1, H)),
      Wl1, jnp.reshape(bl1, (1, Wl1.shape[1])),
      Wl3, jnp.reshape(bl3, (1, C)))


def kernel(x, edge_index, gin_eps, W1, b1, gamma1, beta1, W2, b2,
           gamma4, beta4, Wl1, bl1, Wl3, bl3):
    ei = edge_index.astype(jnp.int32)
    pad = EPP - EPW
    src = jnp.reshape(
        jnp.pad(jnp.reshape(ei[0], (NW, EPW)), ((0, 0), (0, pad))),
        (NW, NCHUNK, CHUNK))
    dst = jnp.reshape(
        jnp.pad(jnp.reshape(ei[1], (NW, EPW)), ((0, 0), (0, pad)),
                constant_values=PAD_ROW),
        (NW, NCHUNK, CHUNK))
    zeros = jnp.zeros((RPT, H), dtype=jnp.float32)
    agg2 = _sc_agg(x, src, dst, zeros)
    return _tc_head(gin_eps, x, agg2, W1, b1, gamma1, beta1, W2, b2,
                    gamma4, beta4, Wl1, bl1, Wl3, bl3)


# speedup vs baseline: 1.0000x; 1.0000x over previous
import functools

import jax
import jax.numpy as jnp
from jax import lax
from jax.experimental import pallas as pl
from jax.experimental.pallas import tpu as pltpu
from jax.experimental.pallas import tpu_sc as plsc

N = 10000
H = 128
E = 320000
NC = 2
NS = 16
NW = NC * NS
EPW = E // NW
CHUNK = 128
NCHUNK = 80
EPP = NCHUNK * CHUNK
NACC = N + 16 * NS
RBUF = 2
SBUF = 4
NOUTER = NCHUNK // SBUF

RPT = (N // NS) // 8 * 8
RTAIL = N - RPT * NS

assert EPW * NW == E and EPP >= EPW
assert NOUTER * SBUF == NCHUNK and RTAIL % 8 == 0


def _make_sc_agg():
    mesh = plsc.VectorSubcoreMesh(core_axis_name="c", subcore_axis_name="s",
                                  num_cores=NC, num_subcores=NS)

    @functools.partial(
        pl.kernel,
        out_type=jax.ShapeDtypeStruct((NC, N, H), jnp.float32),
        mesh=mesh,
        scratch_types=(
            [
                pltpu.VMEM_SHARED((NACC, H), jnp.float32),
                pltpu.VMEM((SBUF, CHUNK), jnp.int32),
                pltpu.VMEM((NCHUNK, CHUNK), jnp.int32),
                pltpu.VMEM((RBUF, CHUNK, H), jnp.float32),
            ]
            + [pltpu.SemaphoreType.DMA for _ in range(SBUF + 2 * RBUF)]
        ),
    )
    def sc_agg(x_hbm, src_hbm, dst_hbm, zeros_hbm, out_hbm,
               acc_sh, sidx_v, dst_v, rows_v, *sems):
        isem = sems[:SBUF]
        gsem = sems[SBUF:SBUF + RBUF]
        ssem = sems[SBUF + RBUF:]
        c = lax.axis_index("c")
        s = lax.axis_index("s")
        wid = s * NC + c

        pltpu.sync_copy(zeros_hbm.at[pl.ds(0, RPT)],
                        acc_sh.at[pl.ds(s * RPT, RPT)])

        @pl.when(s == NS - 1)
        def _():
            pltpu.sync_copy(zeros_hbm.at[pl.ds(0, RTAIL)],
                            acc_sh.at[pl.ds(NS * RPT, RTAIL)])

        pltpu.sync_copy(dst_hbm.at[wid], dst_v)
        plsc.subcore_barrier()

        def start_idx(j, sb):
            pltpu.async_copy(src_hbm.at[wid, j], sidx_v.at[sb], isem[sb])

        def wait_idx(sb):
            pltpu.make_async_copy(src_hbm.at[wid, 0], sidx_v.at[sb],
                                  isem[sb]).wait()

        def start_gather(sb, rb):
            pltpu.async_copy(x_hbm.at[sidx_v.at[sb]], rows_v.at[rb], gsem[rb])

        def wait_gather(rb):
            pltpu.make_async_copy(x_hbm.at[sidx_v.at[0]], rows_v.at[rb],
                                  gsem[rb]).wait()

        def start_scatter(j, rb):
            pltpu.async_copy(rows_v.at[rb], acc_sh.at[dst_v.at[j]], ssem[rb],
                             add=True)

        def wait_scatter(rb):
            pltpu.make_async_copy(rows_v.at[rb], acc_sh.at[dst_v.at[0]],
                                  ssem[rb]).wait()

        for j in range(SBUF - 1):
            start_idx(j, j)
        wait_idx(0)
        start_gather(0, 0)

        def outer(o, carry):
            for b in range(SBUF):
                j = o * SBUF + b

                @pl.when(j + SBUF - 1 < NCHUNK)
                def _():
                    start_idx(j + SBUF - 1, (b + SBUF - 1) % SBUF)

                @pl.when(j + 1 < NCHUNK)
                def _():
                    wait_idx((b + 1) % SBUF)

                    @pl.when(j - 1 >= 0)
                    def _():
                        wait_scatter((b + 1) % RBUF)

                    start_gather((b + 1) % SBUF, (b + 1) % RBUF)

                wait_gather(b % RBUF)
                start_scatter(j, b % RBUF)
            return carry

        lax.fori_loop(0, NOUTER, outer, 0, unroll=False)

        for rb in range(RBUF):
            wait_scatter(rb)

        plsc.subcore_barrier()
        pltpu.sync_copy(acc_sh.at[pl.ds(s * RPT, RPT)],
                        out_hbm.at[c, pl.ds(s * RPT, RPT)])

        @pl.when(s == NS - 1)
        def _():
            pltpu.sync_copy(acc_sh.at[pl.ds(NS * RPT, RTAIL)],
                            out_hbm.at[c, pl.ds(NS * RPT, RTAIL)])

    return sc_agg


_SC_AGG_CACHE = []


def _sc_agg(*args):
    if not _SC_AGG_CACHE:
        _SC_AGG_CACHE.append(_make_sc_agg())
    return _SC_AGG_CACHE[0](*args)


def _tc_head_body(eps_ref, x_ref, agg_ref, w1_ref, b1_ref, g1_ref, be1_ref,
                  w2_ref, b2_ref, g4_ref, be4_ref, wl1_ref, bl1_ref,
                  wl3_ref, bl3_ref, out_ref):
    eps = eps_ref[0, 0]
    x = x_ref[...]
    agg = agg_ref[0] + agg_ref[1]

    h = (1.0 + eps) * x + agg
    h = jnp.dot(h, w1_ref[...], preferred_element_type=jnp.float32) + b1_ref[...]
    m = jnp.mean(h, axis=0, keepdims=True)
    v = jnp.mean((h - m) * (h - m), axis=0, keepdims=True)
    h = g1_ref[...] * (h - m) * lax.rsqrt(v + 1e-5) + be1_ref[...]
    h = jnp.maximum(h, 0.0)
    h = jnp.dot(h, w2_ref[...], preferred_element_type=jnp.float32) + b2_ref[...]
    h = jnp.where(h > 0, h, 0.01 * h)
    m4 = jnp.mean(h, axis=0, keepdims=True)
    v4 = jnp.mean((h - m4) * (h - m4), axis=0, keepdims=True)
    h = g4_ref[...] * (h - m4) * lax.rsqrt(v4 + 1e-5) + be4_ref[...]
    h = jnp.where(h > 0, h, 0.1 * h)
    h = x + 0.01 * h
    h = jnp.dot(h, wl1_ref[...], preferred_element_type=jnp.float32) + bl1_ref[...]
    h = jnp.where(h > 0, h, 0.1 * h)
    out_ref[...] = (jnp.dot(h, wl3_ref[...], preferred_element_type=jnp.float32)
                    + bl3_ref[...])


def _tc_head(gin_eps, x, agg2, W1, b1, gamma1, beta1, W2, b2, gamma4, beta4,
             Wl1, bl1, Wl3, bl3):
    C = Wl3.shape[1]
    eps_arr = jnp.reshape(gin_eps, (1, 1))
    smem_spec = pl.BlockSpec(memory_space=pltpu.SMEM)
    return pl.pallas_call(
        _tc_head_body,
        out_shape=jax.ShapeDtypeStruct((N, C), jnp.float32),
        in_specs=[smem_spec] + [pl.BlockSpec(memory_space=pltpu.VMEM)] * 14,
        out_specs=pl.BlockSpec(memory_space=pltpu.VMEM),
    )(eps_arr, x, agg2,
      W1, jnp.reshape(b1, (1, H)), jnp.reshape(gamma1, (1, H)),
      jnp.reshape(beta1, (1, H)),
      W2, jnp.reshape(b2, (1, H)), jnp.reshape(gamma4, (1, H)),
      jnp.reshape(beta4, (1, H)),
      Wl1, jnp.reshape(bl1, (1, Wl1.shape[1])),
      Wl3, jnp.reshape(bl3, (1, C)))


def kernel(x, edge_index, gin_eps, W1, b1, gamma1, beta1, W2, b2,
           gamma4, beta4, Wl1, bl1, Wl3, bl3):
    ei = edge_index.astype(jnp.int32)
    pad = EPP - EPW
    src = jnp.reshape(
        jnp.pad(jnp.reshape(ei[0], (NW, EPW)), ((0, 0), (0, pad))),
        (NW, NCHUNK, CHUNK))
    w = jnp.arange(NW, dtype=jnp.int32)[:, None]
    i = jnp.arange(pad, dtype=jnp.int32)[None, :]
    padblk = jnp.broadcast_to(N + (w // NC) * 16 + (i % 16), (NW, pad))
    dst = jnp.reshape(
        jnp.concatenate([jnp.reshape(ei[1], (NW, EPW)), padblk], axis=1),
        (NW, NCHUNK, CHUNK))
    zeros = jnp.zeros((RPT, H), dtype=jnp.float32)
    agg2 = _sc_agg(x, src, dst, zeros)
    return _tc_head(gin_eps, x, agg2, W1, b1, gamma1, beta1, W2, b2,
                    gamma4, beta4, Wl1, bl1, Wl3, bl3)


# speedup vs baseline: 2.1534x; 2.1534x over previous
import functools

import jax
import jax.numpy as jnp
from jax import lax
from jax.experimental import pallas as pl
from jax.experimental.pallas import tpu as pltpu
from jax.experimental.pallas import tpu_sc as plsc

N = 10000
H = 128
E = 320000
NC = 2
NS = 16
NW = NC * NS
EPW = E // NW
CHUNK = 80
NCHUNK = EPW // CHUNK

RPT = (N // NS) // 8 * 8
RTAIL = N - RPT * NS

assert EPW * NW == E and NCHUNK * CHUNK == EPW and RTAIL % 8 == 0


def _make_sc_agg():
    mesh = plsc.VectorSubcoreMesh(core_axis_name="c", subcore_axis_name="s",
                                  num_cores=NC, num_subcores=NS)

    @functools.partial(
        pl.kernel,
        out_type=jax.ShapeDtypeStruct((NC, N, H), jnp.float32),
        mesh=mesh,
        scratch_types=(
            [
                pltpu.VMEM_SHARED((N, H), jnp.float32),
                pltpu.VMEM((2, CHUNK, H), jnp.float32),
            ]
            + [pltpu.VMEM((CHUNK,), jnp.int32) for _ in range(4)]
            + [pltpu.SemaphoreType.DMA for _ in range(6)]
        ),
    )
    def sc_agg(x_hbm, src_hbm, dst_hbm, zeros_hbm, out_hbm,
               acc_sh, rows_v, *rest):
        src_v = rest[0:2]
        dst_v = rest[2:4]
        isem_s = rest[4:6]
        isem_d = rest[6:8]
        gsem = rest[8:10]
        c = lax.axis_index("c")
        s = lax.axis_index("s")
        wid = s * NC + c

        pltpu.sync_copy(zeros_hbm.at[pl.ds(0, RPT)],
                        acc_sh.at[pl.ds(s * RPT, RPT)])

        @pl.when(s == NS - 1)
        def _():
            pltpu.sync_copy(zeros_hbm.at[pl.ds(0, RTAIL)],
                            acc_sh.at[pl.ds(NS * RPT, RTAIL)])

        plsc.subcore_barrier()
        base = wid * EPW

        def start_idx(j, p):
            off = base + j * CHUNK
            pltpu.async_copy(src_hbm.at[pl.ds(off, CHUNK)], src_v[p],
                             isem_s[p])
            pltpu.async_copy(dst_hbm.at[pl.ds(off, CHUNK)], dst_v[p],
                             isem_d[p])

        def wait_idx(p):
            pltpu.make_async_copy(src_hbm.at[pl.ds(0, CHUNK)], src_v[p],
                                  isem_s[p]).wait()
            pltpu.make_async_copy(dst_hbm.at[pl.ds(0, CHUNK)], dst_v[p],
                                  isem_d[p]).wait()

        def start_gather(p):
            pltpu.async_copy(x_hbm.at[src_v[p]], rows_v.at[p], gsem[p])

        def wait_gather(p):
            pltpu.make_async_copy(x_hbm.at[src_v[p]], rows_v.at[p],
                                  gsem[p]).wait()

        def scatter(p):
            pltpu.sync_copy(rows_v.at[p], acc_sh.at[dst_v[p]], add=True)

        start_idx(0, 0)
        wait_idx(0)
        start_gather(0)
        start_idx(1, 1)

        def body(j, p):
            wait_gather(p)

            @pl.when(j + 1 < NCHUNK)
            def _():
                wait_idx(1 - p)
                start_gather(1 - p)

            scatter(p)

            @pl.when(j + 2 < NCHUNK)
            def _():
                start_idx(j + 2, p)

        def outer(o, carry):
            for b in range(2):
                body(o * 2 + b, b)
            return carry

        lax.fori_loop(0, NCHUNK // 2, outer, 0, unroll=False)
        body(NCHUNK - 1, (NCHUNK - 1) % 2)

        plsc.subcore_barrier()
        pltpu.sync_copy(acc_sh.at[pl.ds(s * RPT, RPT)],
                        out_hbm.at[c, pl.ds(s * RPT, RPT)])

        @pl.when(s == NS - 1)
        def _():
            pltpu.sync_copy(acc_sh.at[pl.ds(NS * RPT, RTAIL)],
                            out_hbm.at[c, pl.ds(NS * RPT, RTAIL)])

    return sc_agg


_SC_AGG_CACHE = []


def _sc_agg(*args):
    if not _SC_AGG_CACHE:
        _SC_AGG_CACHE.append(_make_sc_agg())
    return _SC_AGG_CACHE[0](*args)


def _tc_head_body(eps_ref, x_ref, agg_ref, w1_ref, b1_ref, g1_ref, be1_ref,
                  w2_ref, b2_ref, g4_ref, be4_ref, wl1_ref, bl1_ref,
                  wl3_ref, bl3_ref, out_ref):
    eps = eps_ref[0, 0]
    x = x_ref[...]
    agg = agg_ref[0] + agg_ref[1]

    h = (1.0 + eps) * x + agg
    h = jnp.dot(h, w1_ref[...], preferred_element_type=jnp.float32) + b1_ref[...]
    m = jnp.mean(h, axis=0, keepdims=True)
    v = jnp.mean((h - m) * (h - m), axis=0, keepdims=True)
    h = g1_ref[...] * (h - m) * lax.rsqrt(v + 1e-5) + be1_ref[...]
    h = jnp.maximum(h, 0.0)
    h = jnp.dot(h, w2_ref[...], preferred_element_type=jnp.float32) + b2_ref[...]
    h = jnp.where(h > 0, h, 0.01 * h)
    m4 = jnp.mean(h, axis=0, keepdims=True)
    v4 = jnp.mean((h - m4) * (h - m4), axis=0, keepdims=True)
    h = g4_ref[...] * (h - m4) * lax.rsqrt(v4 + 1e-5) + be4_ref[...]
    h = jnp.where(h > 0, h, 0.1 * h)
    h = x + 0.01 * h
    h = jnp.dot(h, wl1_ref[...], preferred_element_type=jnp.float32) + bl1_ref[...]
    h = jnp.where(h > 0, h, 0.1 * h)
    out_ref[...] = (jnp.dot(h, wl3_ref[...], preferred_element_type=jnp.float32)
                    + bl3_ref[...])


def _tc_head(gin_eps, x, agg2, W1, b1, gamma1, beta1, W2, b2, gamma4, beta4,
             Wl1, bl1, Wl3, bl3):
    C = Wl3.shape[1]
    eps_arr = jnp.reshape(gin_eps, (1, 1))
    smem_spec = pl.BlockSpec(memory_space=pltpu.SMEM)
    return pl.pallas_call(
        _tc_head_body,
        out_shape=jax.ShapeDtypeStruct((N, C), jnp.float32),
        in_specs=[smem_spec] + [pl.BlockSpec(memory_space=pltpu.VMEM)] * 14,
        out_specs=pl.BlockSpec(memory_space=pltpu.VMEM),
    )(eps_arr, x, agg2,
      W1, jnp.reshape(b1, (1, H)), jnp.reshape(gamma1, (1, H)),
      jnp.reshape(beta1, (1, H)),
      W2, jnp.reshape(b2, (1, H)), jnp.reshape(gamma4, (1, H)),
      jnp.reshape(beta4, (1, H)),
      Wl1, jnp.reshape(bl1, (1, Wl1.shape[1])),
      Wl3, jnp.reshape(bl3, (1, C)))


def kernel(x, edge_index, gin_eps, W1, b1, gamma1, beta1, W2, b2,
           gamma4, beta4, Wl1, bl1, Wl3, bl3):
    ei = edge_index.astype(jnp.int32)
    src = ei[0]
    dst = ei[1]
    zeros = jnp.zeros((RPT, H), dtype=jnp.float32)
    agg2 = _sc_agg(x, src, dst, zeros)
    return _tc_head(gin_eps, x, agg2, W1, b1, gamma1, beta1, W2, b2,
                    gamma4, beta4, Wl1, bl1, Wl3, bl3)
